# Initial kernel scaffold; baseline (speedup 1.0000x reference)
import jax
import jax.numpy as jnp
from jax.experimental import pallas as pl


def kernel(x, edge_index, W1, b1, W2, b2):
    raise NotImplementedError("write your pallas kernel here")


# speedup vs baseline: 47.3612x; 47.3612x over previous
import functools

import jax
import jax.numpy as jnp
from jax import lax
from jax.experimental import pallas as pl
from jax.experimental.pallas import tpu as pltpu
from jax.experimental.pallas import tpu_sc as plsc

N = 10000
N_PAD = 10240
E = 320000
CH = 8
TILES = 32
K = 80
NB = 8
EPT = K * 128
E_PAD = TILES * EPT
DUMMY = N_PAD - 1
SLICE = N_PAD // 16

_mesh = plsc.VectorSubcoreMesh(core_axis_name="c", subcore_axis_name="s")
_HI = lax.Precision.HIGHEST
_SC_PARAMS = pltpu.CompilerParams(use_tc_tiling_on_sc=False)


@functools.partial(
    pl.kernel,
    out_type=jax.ShapeDtypeStruct((2, N_PAD, CH), jnp.float32),
    mesh=_mesh,
    compiler_params=_SC_PARAMS,
    scratch_types=[
        pltpu.VMEM((NB, 128), jnp.int32),
        pltpu.VMEM((128, CH), jnp.float32),
        pltpu.VMEM_SHARED((N_PAD, CH), jnp.float32),
    ],
)
def _deg(dsts, zeros, ones, out, dst_v, ones_v, acc_sh):
    c = lax.axis_index("c")
    s = lax.axis_index("s")
    wid = c * 16 + s
    r0 = s * SLICE
    pltpu.sync_copy(ones, ones_v)
    pltpu.sync_copy(zeros, acc_sh.at[pl.ds(r0, SLICE)])
    plsc.subcore_barrier()

    def body(g, carry):
        pltpu.sync_copy(dsts.at[wid, pl.ds(g * NB, NB)], dst_v)
        for b in range(NB):
            pltpu.sync_copy(ones_v, acc_sh.at[dst_v.at[b]], add=True)
        return carry

    lax.fori_loop(0, K // NB, body, 0)
    plsc.subcore_barrier()
    pltpu.sync_copy(acc_sh.at[pl.ds(r0, SLICE)], out.at[c, pl.ds(r0, SLICE)])


@functools.partial(
    pl.kernel,
    out_type=jax.ShapeDtypeStruct((2, N_PAD, CH), jnp.float32),
    mesh=_mesh,
    compiler_params=_SC_PARAMS,
    scratch_types=[
        pltpu.VMEM((NB, 128), jnp.int32),
        pltpu.VMEM((NB, 128), jnp.int32),
        pltpu.VMEM((NB, 128, CH), jnp.float32),
        pltpu.VMEM_SHARED((N_PAD, CH), jnp.float32),
        pltpu.VMEM_SHARED((N_PAD, CH), jnp.float32),
        pltpu.SemaphoreType.DMA,
    ],
)
def _edge_pass(u_hbm, srcs, dsts, zeros, out,
               src_v, dst_v, rows_v, u_sh, acc_sh, sem):
    c = lax.axis_index("c")
    s = lax.axis_index("s")
    wid = c * 16 + s
    r0 = s * SLICE
    pltpu.sync_copy(u_hbm.at[pl.ds(r0, SLICE)], u_sh.at[pl.ds(r0, SLICE)])
    pltpu.sync_copy(zeros, acc_sh.at[pl.ds(r0, SLICE)])
    plsc.subcore_barrier()

    def body(g, carry):
        pltpu.sync_copy(srcs.at[wid, pl.ds(g * NB, NB)], src_v)
        pltpu.sync_copy(dsts.at[wid, pl.ds(g * NB, NB)], dst_v)
        for b in range(NB):
            pltpu.async_copy(u_sh.at[src_v.at[b]], rows_v.at[b], sem).wait()
        for b in range(NB):
            pltpu.sync_copy(rows_v.at[b], acc_sh.at[dst_v.at[b]], add=True)
        return carry

    lax.fori_loop(0, K // NB, body, 0)
    plsc.subcore_barrier()
    pltpu.sync_copy(acc_sh.at[pl.ds(r0, SLICE)], out.at[c, pl.ds(r0, SLICE)])


def _tc1_body(x_ref, w1_ref, w2_ref, degp_ref, u1_ref, dinv_ref):
    w12 = jnp.dot(w1_ref[...], w2_ref[...],
                  preferred_element_type=jnp.float32, precision=_HI)
    y = jnp.dot(x_ref[...], w12,
                preferred_element_type=jnp.float32, precision=_HI)
    col = lax.broadcasted_iota(jnp.int32, y.shape, 1)
    y8 = jnp.where(col == 3, 1.0, y)
    degp = degp_ref[...]
    dinv = lax.rsqrt(degp[0] + degp[1] + 1.0)
    dinv_ref[...] = dinv
    u1_ref[...] = dinv * y8


_tc1 = pl.pallas_call(
    _tc1_body,
    out_shape=(
        jax.ShapeDtypeStruct((N_PAD, CH), jnp.float32),
        jax.ShapeDtypeStruct((N_PAD, CH), jnp.float32),
    ),
)


def _mid2_body(dinv_ref, u1_ref, a1_ref, u2_ref):
    a1 = a1_ref[...]
    dinv = dinv_ref[...]
    z1 = dinv * (a1[0] + a1[1] + u1_ref[...])
    u2_ref[...] = dinv * z1


_mid2 = pl.pallas_call(
    _mid2_body,
    out_shape=jax.ShapeDtypeStruct((N_PAD, CH), jnp.float32),
)


def _final_body(dinv_ref, u1_ref, a1_ref, a2_ref, b1_ref, w2_ref, b2_ref,
                out_ref):
    dinv = dinv_ref[...]
    a1 = a1_ref[...]
    a2 = a2_ref[...]
    z1 = dinv * (a1[0] + a1[1] + u1_ref[...])
    u2 = dinv * z1
    z2 = dinv * (a2[0] + a2[1] + u2)
    cvec = jnp.dot(b1_ref[...], w2_ref[...],
                   preferred_element_type=jnp.float32, precision=_HI)
    res = z2 + z1[:, 3:4] * cvec + b2_ref[...]
    out_ref[...] = res[:N, :3]


_final = pl.pallas_call(
    _final_body,
    out_shape=jax.ShapeDtypeStruct((N, 3), jnp.float32),
)


def kernel(x, edge_index, W1, b1, W2, b2):
    x_pad = jnp.pad(x, ((0, N_PAD - N), (0, 0)))
    W2pad = jnp.pad(W2, ((0, 0), (0, CH - 3)))
    src = edge_index[0].astype(jnp.int32)
    dst = edge_index[1].astype(jnp.int32)
    pad = jnp.full((E_PAD - E,), DUMMY, dtype=jnp.int32)
    srcs = jnp.concatenate([src, pad]).reshape(TILES, K, 128)
    dsts = jnp.concatenate([dst, pad]).reshape(TILES, K, 128)
    zeros = jnp.zeros((SLICE, CH), jnp.float32)
    ones = jnp.ones((128, CH), jnp.float32)

    degp = _deg(dsts, zeros, ones)
    u1, dinv = _tc1(x_pad, W1, W2pad, degp)
    acc1 = _edge_pass(u1, srcs, dsts, zeros)
    u2 = _mid2(dinv, u1, acc1)
    acc2 = _edge_pass(u2, srcs, dsts, zeros)
    return _final(
        dinv, u1, acc1, acc2,
        b1.reshape(1, 128), W2pad, jnp.pad(b2, (0, CH - 3)).reshape(1, CH),
    )


# speedup vs baseline: 53.9025x; 1.1381x over previous
import functools

import jax
import jax.numpy as jnp
from jax import lax
from jax.experimental import pallas as pl
from jax.experimental.pallas import tpu as pltpu
from jax.experimental.pallas import tpu_sc as plsc

N = 10000
N_PAD = 10240
E = 320000
CH = 8
TILES = 32
K = 80
NB = 8
EPT = K * 128
E_PAD = TILES * EPT
DUMMY = N_PAD - 1
SLICE = N_PAD // 16

_mesh = plsc.VectorSubcoreMesh(core_axis_name="c", subcore_axis_name="s")
_HI = lax.Precision.HIGHEST
_SC_PARAMS = pltpu.CompilerParams(use_tc_tiling_on_sc=False)


@functools.partial(
    pl.kernel,
    out_type=jax.ShapeDtypeStruct((2, N_PAD, CH), jnp.float32),
    mesh=_mesh,
    compiler_params=_SC_PARAMS,
    scratch_types=[
        pltpu.VMEM((NB, 128), jnp.int32),
        pltpu.VMEM((128, CH), jnp.float32),
        pltpu.VMEM_SHARED((N_PAD, CH), jnp.float32),
        pltpu.SemaphoreType.DMA,
    ],
)
def _deg(dsts, zeros, ones, out, dst_v, ones_v, acc_sh, sem):
    c = lax.axis_index("c")
    s = lax.axis_index("s")
    wid = c * 16 + s
    r0 = s * SLICE
    pltpu.sync_copy(ones, ones_v)
    pltpu.sync_copy(zeros, acc_sh.at[pl.ds(r0, SLICE)])
    plsc.subcore_barrier()

    def body(g, carry):
        pltpu.sync_copy(dsts.at[wid, pl.ds(g * NB, NB)], dst_v)
        cps = [
            pltpu.async_copy(ones_v, acc_sh.at[dst_v.at[b]], sem, add=True)
            for b in range(NB)
        ]
        for cp in cps:
            cp.wait()
        return carry

    lax.fori_loop(0, K // NB, body, 0)
    plsc.subcore_barrier()
    pltpu.sync_copy(acc_sh.at[pl.ds(r0, SLICE)], out.at[c, pl.ds(r0, SLICE)])


@functools.partial(
    pl.kernel,
    out_type=jax.ShapeDtypeStruct((2, N_PAD, CH), jnp.float32),
    mesh=_mesh,
    compiler_params=_SC_PARAMS,
    scratch_types=[
        pltpu.VMEM((NB, 128), jnp.int32),
        pltpu.VMEM((NB, 128), jnp.int32),
        pltpu.VMEM((NB, 128, CH), jnp.float32),
        pltpu.VMEM_SHARED((N_PAD, CH), jnp.float32),
        pltpu.VMEM_SHARED((N_PAD, CH), jnp.float32),
        pltpu.SemaphoreType.DMA,
        pltpu.SemaphoreType.DMA,
    ],
)
def _edge_pass(u_hbm, srcs, dsts, zeros, out,
               src_v, dst_v, rows_v, u_sh, acc_sh, sem, sem2):
    c = lax.axis_index("c")
    s = lax.axis_index("s")
    wid = c * 16 + s
    r0 = s * SLICE
    pltpu.sync_copy(u_hbm.at[pl.ds(r0, SLICE)], u_sh.at[pl.ds(r0, SLICE)])
    pltpu.sync_copy(zeros, acc_sh.at[pl.ds(r0, SLICE)])
    plsc.subcore_barrier()

    def body(g, carry):
        pltpu.sync_copy(srcs.at[wid, pl.ds(g * NB, NB)], src_v)
        pltpu.sync_copy(dsts.at[wid, pl.ds(g * NB, NB)], dst_v)
        gcps = [
            pltpu.async_copy(u_sh.at[src_v.at[b]], rows_v.at[b], sem)
            for b in range(NB)
        ]
        scps = []
        for b in range(NB):
            gcps[b].wait()
            scps.append(
                pltpu.async_copy(rows_v.at[b], acc_sh.at[dst_v.at[b]],
                                 sem2, add=True))
        for cp in scps:
            cp.wait()
        return carry

    lax.fori_loop(0, K // NB, body, 0)
    plsc.subcore_barrier()
    pltpu.sync_copy(acc_sh.at[pl.ds(r0, SLICE)], out.at[c, pl.ds(r0, SLICE)])


def _tc1_body(x_ref, w1_ref, w2_ref, degp_ref, u1_ref, dinv_ref):
    w12 = jnp.dot(w1_ref[...], w2_ref[...],
                  preferred_element_type=jnp.float32, precision=_HI)
    y = jnp.dot(x_ref[...], w12, preferred_element_type=jnp.float32)
    col = lax.broadcasted_iota(jnp.int32, y.shape, 1)
    y8 = jnp.where(col == 3, 1.0, y)
    degp = degp_ref[...]
    dinv = lax.rsqrt(degp[0] + degp[1] + 1.0)
    dinv_ref[...] = dinv
    u1_ref[:N] = dinv[:N] * y8
    u1_ref[pl.ds(N, N_PAD - N)] = jnp.zeros((N_PAD - N, CH), jnp.float32)


_tc1 = pl.pallas_call(
    _tc1_body,
    out_shape=(
        jax.ShapeDtypeStruct((N_PAD, CH), jnp.float32),
        jax.ShapeDtypeStruct((N_PAD, CH), jnp.float32),
    ),
)


def _mid2_body(dinv_ref, u1_ref, a1_ref, u2_ref):
    a1 = a1_ref[...]
    dinv = dinv_ref[...]
    z1 = dinv * (a1[0] + a1[1] + u1_ref[...])
    u2_ref[...] = dinv * z1


_mid2 = pl.pallas_call(
    _mid2_body,
    out_shape=jax.ShapeDtypeStruct((N_PAD, CH), jnp.float32),
)


def _final_body(dinv_ref, u1_ref, a1_ref, a2_ref, b1_ref, w2_ref, b2_ref,
                out_ref):
    dinv = dinv_ref[...]
    a1 = a1_ref[...]
    a2 = a2_ref[...]
    z1 = dinv * (a1[0] + a1[1] + u1_ref[...])
    u2 = dinv * z1
    z2 = dinv * (a2[0] + a2[1] + u2)
    cvec = jnp.dot(b1_ref[...], w2_ref[...],
                   preferred_element_type=jnp.float32, precision=_HI)
    res = z2 + z1[:, 3:4] * cvec + b2_ref[...]
    out_ref[...] = res[:N, :3]


_final = pl.pallas_call(
    _final_body,
    out_shape=jax.ShapeDtypeStruct((N, 3), jnp.float32),
)


def kernel(x, edge_index, W1, b1, W2, b2):
    W2pad = jnp.pad(W2, ((0, 0), (0, CH - 3)))
    src = edge_index[0].astype(jnp.int32)
    dst = edge_index[1].astype(jnp.int32)
    pad = jnp.full((E_PAD - E,), DUMMY, dtype=jnp.int32)
    srcs = jnp.concatenate([src, pad]).reshape(TILES, K, 128)
    dsts = jnp.concatenate([dst, pad]).reshape(TILES, K, 128)
    zeros = jnp.zeros((SLICE, CH), jnp.float32)
    ones = jnp.ones((128, CH), jnp.float32)

    degp = _deg(dsts, zeros, ones)
    u1, dinv = _tc1(x, W1, W2pad, degp)
    acc1 = _edge_pass(u1, srcs, dsts, zeros)
    u2 = _mid2(dinv, u1, acc1)
    acc2 = _edge_pass(u2, srcs, dsts, zeros)
    return _final(
        dinv, u1, acc1, acc2,
        b1.reshape(1, 128), W2pad, jnp.pad(b2, (0, CH - 3)).reshape(1, CH),
    )


# speedup vs baseline: 59.4427x; 1.1028x over previous
import functools

import jax
import jax.numpy as jnp
from jax import lax
from jax.experimental import pallas as pl
from jax.experimental.pallas import tpu as pltpu
from jax.experimental.pallas import tpu_sc as plsc

N = 10000
N_PAD = 10240
E = 320000
CH = 8
TILES = 32
NCHUNK = E // 128
CPT = NCHUNK // TILES
NXTRA = NCHUNK - CPT * TILES
NB = 13
SLICE = N_PAD // 16

_mesh = plsc.VectorSubcoreMesh(core_axis_name="c", subcore_axis_name="s")
_HI = lax.Precision.HIGHEST
_SC_PARAMS = pltpu.CompilerParams(use_tc_tiling_on_sc=False)


@functools.partial(
    pl.kernel,
    out_type=jax.ShapeDtypeStruct((2, N_PAD, CH), jnp.float32),
    mesh=_mesh,
    compiler_params=_SC_PARAMS,
    scratch_types=[
        pltpu.VMEM((NB, 128), jnp.int32),
        pltpu.VMEM((128, CH), jnp.float32),
        pltpu.VMEM_SHARED((N_PAD, CH), jnp.float32),
        pltpu.SemaphoreType.DMA,
    ],
)
def _deg(dsts, zeros, ones, out, dst_v, ones_v, acc_sh, sem):
    c = lax.axis_index("c")
    s = lax.axis_index("s")
    wid = c * 16 + s
    r0 = s * SLICE
    base = wid * CPT
    pltpu.sync_copy(ones, ones_v)
    pltpu.sync_copy(zeros, acc_sh.at[pl.ds(r0, SLICE)])
    plsc.subcore_barrier()

    def body(g, carry):
        pltpu.sync_copy(dsts.at[pl.ds(base + g * NB, NB)], dst_v)
        cps = [
            pltpu.async_copy(ones_v, acc_sh.at[dst_v.at[b]], sem, add=True)
            for b in range(NB)
        ]
        for cp in cps:
            cp.wait()
        return carry

    lax.fori_loop(0, CPT // NB, body, 0)

    @pl.when(wid < NXTRA)
    def _():
        pltpu.sync_copy(dsts.at[pl.ds(TILES * CPT + wid, 1)],
                        dst_v.at[pl.ds(0, 1)])
        pltpu.async_copy(ones_v, acc_sh.at[dst_v.at[0]], sem, add=True).wait()

    plsc.subcore_barrier()
    pltpu.sync_copy(acc_sh.at[pl.ds(r0, SLICE)], out.at[c, pl.ds(r0, SLICE)])


@functools.partial(
    pl.kernel,
    out_type=jax.ShapeDtypeStruct((2, N_PAD, CH), jnp.float32),
    mesh=_mesh,
    compiler_params=_SC_PARAMS,
    scratch_types=[
        pltpu.VMEM((NB, 128), jnp.int32),
        pltpu.VMEM((NB, 128), jnp.int32),
        pltpu.VMEM((NB, 128, CH), jnp.float32),
        pltpu.VMEM_SHARED((N_PAD, CH), jnp.float32),
        pltpu.VMEM_SHARED((N_PAD, CH), jnp.float32),
        pltpu.SemaphoreType.DMA,
        pltpu.SemaphoreType.DMA,
    ],
)
def _edge_pass(u_hbm, srcs, dsts, zeros, out,
               src_v, dst_v, rows_v, u_sh, acc_sh, sem, sem2):
    c = lax.axis_index("c")
    s = lax.axis_index("s")
    wid = c * 16 + s
    r0 = s * SLICE
    base = wid * CPT
    pltpu.sync_copy(u_hbm.at[pl.ds(r0, SLICE)], u_sh.at[pl.ds(r0, SLICE)])
    pltpu.sync_copy(zeros, acc_sh.at[pl.ds(r0, SLICE)])
    plsc.subcore_barrier()

    def body(g, carry):
        pltpu.sync_copy(srcs.at[pl.ds(base + g * NB, NB)], src_v)
        pltpu.sync_copy(dsts.at[pl.ds(base + g * NB, NB)], dst_v)
        gcps = [
            pltpu.async_copy(u_sh.at[src_v.at[b]], rows_v.at[b], sem)
            for b in range(NB)
        ]
        scps = []
        for b in range(NB):
            gcps[b].wait()
            scps.append(
                pltpu.async_copy(rows_v.at[b], acc_sh.at[dst_v.at[b]],
                                 sem2, add=True))
        for cp in scps:
            cp.wait()
        return carry

    lax.fori_loop(0, CPT // NB, body, 0)

    @pl.when(wid < NXTRA)
    def _():
        pltpu.sync_copy(srcs.at[pl.ds(TILES * CPT + wid, 1)],
                        src_v.at[pl.ds(0, 1)])
        pltpu.sync_copy(dsts.at[pl.ds(TILES * CPT + wid, 1)],
                        dst_v.at[pl.ds(0, 1)])
        pltpu.async_copy(u_sh.at[src_v.at[0]], rows_v.at[0], sem).wait()
        pltpu.async_copy(rows_v.at[0], acc_sh.at[dst_v.at[0]],
                         sem2, add=True).wait()

    plsc.subcore_barrier()
    pltpu.sync_copy(acc_sh.at[pl.ds(r0, SLICE)], out.at[c, pl.ds(r0, SLICE)])


def _mm_body(x_ref, w1_ref, w2_ref, y8_ref):
    w12 = jnp.dot(w1_ref[...], w2_ref[...],
                  preferred_element_type=jnp.float32, precision=_HI)
    y = jnp.dot(x_ref[...], w12, preferred_element_type=jnp.float32)
    col = lax.broadcasted_iota(jnp.int32, y.shape, 1)
    y8_ref[...] = jnp.where(col == 3, 1.0, y)


_mm = pl.pallas_call(
    _mm_body,
    out_shape=jax.ShapeDtypeStruct((N, CH), jnp.float32),
)


def _scale_body(y8_ref, degp_ref, u1_ref, dinv_ref):
    degp = degp_ref[...]
    dinv = lax.rsqrt(degp[0] + degp[1] + 1.0)
    dinv_ref[...] = dinv
    u1_ref[:N] = dinv[:N] * y8_ref[...]
    u1_ref[pl.ds(N, N_PAD - N)] = jnp.zeros((N_PAD - N, CH), jnp.float32)


_scale = pl.pallas_call(
    _scale_body,
    out_shape=(
        jax.ShapeDtypeStruct((N_PAD, CH), jnp.float32),
        jax.ShapeDtypeStruct((N_PAD, CH), jnp.float32),
    ),
)


def _mid2_body(dinv_ref, u1_ref, a1_ref, u2_ref):
    a1 = a1_ref[...]
    dinv = dinv_ref[...]
    z1 = dinv * (a1[0] + a1[1] + u1_ref[...])
    u2_ref[...] = dinv * z1


_mid2 = pl.pallas_call(
    _mid2_body,
    out_shape=jax.ShapeDtypeStruct((N_PAD, CH), jnp.float32),
)


def _final_body(dinv_ref, u1_ref, a1_ref, a2_ref, b1_ref, w2_ref, b2_ref,
                out_ref):
    dinv = dinv_ref[...]
    a1 = a1_ref[...]
    a2 = a2_ref[...]
    z1 = dinv * (a1[0] + a1[1] + u1_ref[...])
    u2 = dinv * z1
    z2 = dinv * (a2[0] + a2[1] + u2)
    cvec = jnp.dot(b1_ref[...], w2_ref[...],
                   preferred_element_type=jnp.float32, precision=_HI)
    res = z2 + z1[:, 3:4] * cvec + b2_ref[...]
    out_ref[...] = res[:N, :3]


_final = pl.pallas_call(
    _final_body,
    out_shape=jax.ShapeDtypeStruct((N, 3), jnp.float32),
)


def kernel(x, edge_index, W1, b1, W2, b2):
    W2pad = jnp.pad(W2, ((0, 0), (0, CH - 3)))
    srcs = edge_index[0].astype(jnp.int32).reshape(NCHUNK, 128)
    dsts = edge_index[1].astype(jnp.int32).reshape(NCHUNK, 128)
    zeros = jnp.zeros((SLICE, CH), jnp.float32)
    ones = jnp.ones((128, CH), jnp.float32)

    y8 = _mm(x, W1, W2pad)
    degp = _deg(dsts, zeros, ones)
    u1, dinv = _scale(y8, degp)
    acc1 = _edge_pass(u1, srcs, dsts, zeros)
    u2 = _mid2(dinv, u1, acc1)
    acc2 = _edge_pass(u2, srcs, dsts, zeros)
    return _final(
        dinv, u1, acc1, acc2,
        b1.reshape(1, 128), W2pad, jnp.pad(b2, (0, CH - 3)).reshape(1, CH),
    )


# speedup vs baseline: 77.9905x; 1.3120x over previous
import functools

import jax
import jax.numpy as jnp
from jax import lax
from jax.experimental import pallas as pl
from jax.experimental.pallas import tpu as pltpu
from jax.experimental.pallas import tpu_sc as plsc

N = 10000
N_PAD = 10240
E = 320000
CH = 8
TILES = 32
NCHUNK = E // 128
CPT = NCHUNK // TILES
NXTRA = NCHUNK - CPT * TILES
NB = 13
SLICE = N_PAD // 16

_mesh = plsc.VectorSubcoreMesh(core_axis_name="c", subcore_axis_name="s")
_HI = lax.Precision.HIGHEST
_SC_PARAMS = pltpu.CompilerParams(use_tc_tiling_on_sc=False)


@functools.partial(
    pl.kernel,
    out_type=jax.ShapeDtypeStruct((2, N_PAD, CH), jnp.float32),
    mesh=_mesh,
    compiler_params=_SC_PARAMS,
    scratch_types=[
        pltpu.VMEM((NB, 128), jnp.int32),
        pltpu.VMEM((128, CH), jnp.float32),
        pltpu.VMEM_SHARED((N_PAD, CH), jnp.float32),
        pltpu.SemaphoreType.DMA,
    ],
)
def _deg(dsts, zeros, ones, out, dst_v, ones_v, acc_sh, sem):
    c = lax.axis_index("c")
    s = lax.axis_index("s")
    wid = c * 16 + s
    r0 = s * SLICE
    base = wid * CPT
    pltpu.sync_copy(ones, ones_v)
    pltpu.sync_copy(zeros, acc_sh.at[pl.ds(r0, SLICE)])
    plsc.subcore_barrier()

    def body(g, carry):
        pltpu.sync_copy(dsts.at[pl.ds(base + g * NB, NB)], dst_v)
        cps = [
            pltpu.async_copy(ones_v, acc_sh.at[dst_v.at[b]], sem, add=True)
            for b in range(NB)
        ]
        for cp in cps:
            cp.wait()
        return carry

    lax.fori_loop(0, CPT // NB, body, 0)

    @pl.when(wid < NXTRA)
    def _():
        pltpu.sync_copy(dsts.at[pl.ds(TILES * CPT + wid, 1)],
                        dst_v.at[pl.ds(0, 1)])
        pltpu.async_copy(ones_v, acc_sh.at[dst_v.at[0]], sem, add=True).wait()

    plsc.subcore_barrier()
    pltpu.sync_copy(acc_sh.at[pl.ds(r0, SLICE)], out.at[c, pl.ds(r0, SLICE)])


@functools.partial(
    pl.kernel,
    out_type=jax.ShapeDtypeStruct((2, N_PAD, CH), jnp.float32),
    mesh=_mesh,
    compiler_params=_SC_PARAMS,
    scratch_types=[
        pltpu.VMEM((NB, 128), jnp.int32),
        pltpu.VMEM((NB, 128), jnp.int32),
        pltpu.VMEM((NB, 128, CH), jnp.float32),
        pltpu.VMEM_SHARED((N_PAD, CH), jnp.float32),
        pltpu.VMEM_SHARED((N_PAD, CH), jnp.float32),
        pltpu.SemaphoreType.DMA,
        pltpu.SemaphoreType.DMA,
    ],
)
def _edge_pass(u_hbm, srcs, dsts, zeros, out,
               src_v, dst_v, rows_v, u_sh, acc_sh, sem, sem2):
    c = lax.axis_index("c")
    s = lax.axis_index("s")
    wid = c * 16 + s
    r0 = s * SLICE
    base = wid * CPT
    pltpu.sync_copy(u_hbm.at[pl.ds(r0, SLICE)], u_sh.at[pl.ds(r0, SLICE)])
    pltpu.sync_copy(zeros, acc_sh.at[pl.ds(r0, SLICE)])
    plsc.subcore_barrier()

    def body(g, carry):
        pltpu.sync_copy(srcs.at[pl.ds(base + g * NB, NB)], src_v)
        pltpu.sync_copy(dsts.at[pl.ds(base + g * NB, NB)], dst_v)
        gcps = [
            pltpu.async_copy(u_sh.at[src_v.at[b]], rows_v.at[b], sem)
            for b in range(NB)
        ]
        scps = []
        for b in range(NB):
            gcps[b].wait()
            scps.append(
                pltpu.async_copy(rows_v.at[b], acc_sh.at[dst_v.at[b]],
                                 sem2, add=True))
        for cp in scps:
            cp.wait()
        return carry

    lax.fori_loop(0, CPT // NB, body, 0)

    @pl.when(wid < NXTRA)
    def _():
        pltpu.sync_copy(srcs.at[pl.ds(TILES * CPT + wid, 1)],
                        src_v.at[pl.ds(0, 1)])
        pltpu.sync_copy(dsts.at[pl.ds(TILES * CPT + wid, 1)],
                        dst_v.at[pl.ds(0, 1)])
        pltpu.async_copy(u_sh.at[src_v.at[0]], rows_v.at[0], sem).wait()
        pltpu.async_copy(rows_v.at[0], acc_sh.at[dst_v.at[0]],
                         sem2, add=True).wait()

    plsc.subcore_barrier()
    pltpu.sync_copy(acc_sh.at[pl.ds(r0, SLICE)], out.at[c, pl.ds(r0, SLICE)])


def _mm_body(x_ref, w1_ref, w2_ref, y8_ref):
    w12 = jnp.dot(w1_ref[...], w2_ref[...],
                  preferred_element_type=jnp.float32, precision=_HI)
    y = jnp.dot(x_ref[...], w12, preferred_element_type=jnp.float32)
    col = lax.broadcasted_iota(jnp.int32, y.shape, 1)
    y8_ref[...] = jnp.where(col == 3, 1.0, y)


_mm = pl.pallas_call(
    _mm_body,
    out_shape=jax.ShapeDtypeStruct((N, CH), jnp.float32),
)


FR = N_PAD * CH // 128
FRN = N * CH // 128


def _scale_body(y8_ref, degp_ref, u1_ref, dinv_ref):
    degp = degp_ref[...]
    dinv = lax.rsqrt(degp[0] + degp[1] + 1.0)
    dinv_ref[...] = dinv
    u1_ref[:FRN] = dinv[:FRN] * y8_ref[...]
    u1_ref[pl.ds(FRN, FR - FRN)] = jnp.zeros((FR - FRN, 128), jnp.float32)


_scale = pl.pallas_call(
    _scale_body,
    out_shape=(
        jax.ShapeDtypeStruct((FR, 128), jnp.float32),
        jax.ShapeDtypeStruct((FR, 128), jnp.float32),
    ),
)


def _mid2_body(dinv_ref, u1_ref, a1_ref, u2_ref):
    a1 = a1_ref[...]
    dinv = dinv_ref[...]
    z1 = dinv * (a1[0] + a1[1] + u1_ref[...])
    u2_ref[...] = dinv * z1


_mid2 = pl.pallas_call(
    _mid2_body,
    out_shape=jax.ShapeDtypeStruct((FR, 128), jnp.float32),
)


def _final_body(dinv_ref, u1_ref, a1_ref, a2_ref, b1_ref, w2_ref, b2_ref,
                out_ref):
    dinv = dinv_ref[...]
    a1 = a1_ref[...]
    a2 = a2_ref[...]
    z1 = dinv * (a1[0] + a1[1] + u1_ref[...])
    u2 = dinv * z1
    z2 = dinv * (a2[0] + a2[1] + u2)
    cvec = jnp.dot(b1_ref[...], w2_ref[...],
                   preferred_element_type=jnp.float32, precision=_HI)
    res = z2 + z1[:, 3:4] * cvec + b2_ref[...]
    out_ref[...] = res[:N, :3]


_final = pl.pallas_call(
    _final_body,
    out_shape=jax.ShapeDtypeStruct((N, 3), jnp.float32),
)


def kernel(x, edge_index, W1, b1, W2, b2):
    W2pad = jnp.pad(W2, ((0, 0), (0, CH - 3)))
    srcs = edge_index[0].astype(jnp.int32).reshape(NCHUNK, 128)
    dsts = edge_index[1].astype(jnp.int32).reshape(NCHUNK, 128)
    zeros = jnp.zeros((SLICE, CH), jnp.float32)
    ones = jnp.ones((128, CH), jnp.float32)

    y8 = _mm(x, W1, W2pad)
    degp = _deg(dsts, zeros, ones)
    u1f, dinvf = _scale(y8.reshape(FRN, 128), degp.reshape(2, FR, 128))
    u1 = u1f.reshape(N_PAD, CH)
    acc1 = _edge_pass(u1, srcs, dsts, zeros)
    u2f = _mid2(dinvf, u1f, acc1.reshape(2, FR, 128))
    acc2 = _edge_pass(u2f.reshape(N_PAD, CH), srcs, dsts, zeros)
    return _final(
        dinvf.reshape(N_PAD, CH), u1, acc1, acc2,
        b1.reshape(1, 128), W2pad, jnp.pad(b2, (0, CH - 3)).reshape(1, CH),
    )
